# baseline (device time: 21147 ns/iter reference)
import jax
import jax.numpy as jnp
from jax import lax
from jax.experimental import pallas as pl
from jax.experimental.pallas import tpu as pltpu

N_DEV = 16
N_TOK = 1024
D_IN = 512
D_OUT = 1024
N_EXP = 64
E_LOCAL = 4
CAP = 12
ROWS = N_TOK // N_DEV
NC = E_LOCAL * CAP


def kernel(x, router_W, route_idx, expert_W):
    del router_W

    def body(x_ref, idx_ref, w_ref, out_ref, c_ref, stage_ref, meta_ref,
             smem_ref, pos_ref, copy_sem, send_sem, recv_sem):
        my = lax.axis_index("i")

        barrier_sem = pltpu.get_barrier_semaphore()
        for d in range(N_DEV):
            pl.semaphore_signal(
                barrier_sem, inc=1,
                device_id=(d,), device_id_type=pl.DeviceIdType.MESH,
            )
        pl.semaphore_wait(barrier_sem, N_DEV)

        r = idx_ref[:, :]
        cols = lax.broadcasted_iota(jnp.int32, (N_TOK, N_EXP), 1)
        onehot = (r == cols).astype(jnp.bfloat16)
        ri = lax.broadcasted_iota(jnp.int32, (N_TOK, N_TOK), 0)
        ci = lax.broadcasted_iota(jnp.int32, (N_TOK, N_TOK), 1)
        tril = (ci < ri).astype(jnp.bfloat16)
        pos = jnp.dot(tril, onehot, preferred_element_type=jnp.float32)
        pos_tok = jnp.sum(pos * onehot.astype(jnp.float32), axis=1,
                          keepdims=True)
        pos_ref[:, :] = pos_tok

        s_le = lax.broadcasted_iota(jnp.int32, (N_TOK, NC), 1) // CAP
        s_p = lax.broadcasted_iota(jnp.int32, (N_TOK, NC), 1) % CAP
        pt = jnp.logical_and(r == my * E_LOCAL + s_le,
                             pos_tok == s_p.astype(jnp.float32))
        ptb = pt.astype(jnp.bfloat16)
        xb = x_ref[:, :].astype(jnp.bfloat16)
        xg = lax.dot_general(
            ptb, xb, (((0,), (0,)), ((), ())),
            preferred_element_type=jnp.float32).astype(jnp.bfloat16)
        for le in range(E_LOCAL):
            c_blk = jnp.dot(
                xg[le * CAP:(le + 1) * CAP, :],
                w_ref[le].astype(jnp.bfloat16),
                preferred_element_type=jnp.float32)
            c_ref[pl.ds(le * CAP, CAP), :, :] = (
                c_blk.astype(jnp.bfloat16)[:, None, :])

        tok_iota = lax.broadcasted_iota(jnp.int32, (N_TOK, 1), 0)
        tok_cols = jnp.concatenate(
            [tok_iota // 32, tok_iota % 32,
             jnp.ones((N_TOK, 1), jnp.int32)], axis=1).astype(jnp.bfloat16)
        hlv = lax.dot_general(
            ptb, tok_cols, (((0,), (0,)), ((), ())),
            preferred_element_type=jnp.float32)
        t_vec = hlv[:, 0:1] * 32.0 + hlv[:, 1:2]
        meta_ref[:, :] = jnp.concatenate(
            [t_vec, hlv[:, 2:3]], axis=1).astype(jnp.int32)
        meta_copy = pltpu.make_async_copy(meta_ref, smem_ref, copy_sem)
        meta_copy.start()
        meta_copy.wait()

        for s in range(NC):
            t = smem_ref[s, 0]
            valid = smem_ref[s, 1]

            @pl.when(valid > 0)
            def _(s=s, t=t):
                dst_rank = lax.div(t, ROWS)
                dst_slot = lax.rem(t, ROWS)
                rdma = pltpu.make_async_remote_copy(
                    src_ref=c_ref.at[s],
                    dst_ref=stage_ref.at[dst_slot],
                    send_sem=send_sem,
                    recv_sem=recv_sem,
                    device_id=(dst_rank,),
                    device_id_type=pl.DeviceIdType.MESH,
                )
                rdma.start()

        pos_mine = pos_ref[pl.ds(my * ROWS, ROWS), :]
        keep_mine = (pos_mine < CAP).astype(jnp.float32)
        n_expect = jnp.sum(keep_mine).astype(jnp.int32)

        dummy = pltpu.make_async_remote_copy(
            src_ref=c_ref.at[0],
            dst_ref=stage_ref.at[0],
            send_sem=send_sem,
            recv_sem=recv_sem,
            device_id=(my,),
            device_id_type=pl.DeviceIdType.MESH,
        )

        def _wait_one(_, carry):
            dummy.wait_recv()
            return carry

        lax.fori_loop(0, n_expect, _wait_one, 0)

        out_ref[:, :] = jnp.where(
            keep_mine > 0, stage_ref[:, 0, :].astype(jnp.float32), 0.0)

        for s in range(NC):
            valid = smem_ref[s, 1]

            @pl.when(valid > 0)
            def _():
                dummy.wait_send()

    return pl.pallas_call(
        body,
        out_shape=jax.ShapeDtypeStruct((ROWS, D_OUT), jnp.float32),
        in_specs=[
            pl.BlockSpec(memory_space=pltpu.VMEM),
            pl.BlockSpec(memory_space=pltpu.VMEM),
            pl.BlockSpec(memory_space=pltpu.VMEM),
        ],
        out_specs=pl.BlockSpec(memory_space=pltpu.VMEM),
        scratch_shapes=[
            pltpu.VMEM((NC, 1, D_OUT), jnp.bfloat16),
            pltpu.VMEM((ROWS, 1, D_OUT), jnp.bfloat16),
            pltpu.VMEM((NC, 2), jnp.int32),
            pltpu.SMEM((NC, 2), jnp.int32),
            pltpu.VMEM((N_TOK, 1), jnp.float32),
            pltpu.SemaphoreType.DMA,
            pltpu.SemaphoreType.DMA,
            pltpu.SemaphoreType.DMA,
        ],
        compiler_params=pltpu.CompilerParams(collective_id=0),
    )(x, route_idx, expert_W)


# device time: 20250 ns/iter; 1.0443x vs baseline; 1.0443x over previous
import jax
import jax.numpy as jnp
from jax import lax
from jax.experimental import pallas as pl
from jax.experimental.pallas import tpu as pltpu

N_DEV = 16
N_TOK = 1024
D_IN = 512
D_OUT = 1024
N_EXP = 64
E_LOCAL = 4
CAP = 12
ROWS = N_TOK // N_DEV
NC = E_LOCAL * CAP


def kernel(x, router_W, route_idx, expert_W):
    del router_W

    def body(x_ref, idx_ref, w_ref, out_ref, c_ref, stage_ref, meta_ref,
             smem_ref, pos_ref, copy_sem, send_sem, recv_sem):
        my = lax.axis_index("i")

        barrier_sem = pltpu.get_barrier_semaphore()
        for d in range(N_DEV):
            pl.semaphore_signal(
                barrier_sem, inc=1,
                device_id=(d,), device_id_type=pl.DeviceIdType.MESH,
            )
        pl.semaphore_wait(barrier_sem, N_DEV)

        r = idx_ref[:, :]
        cols = lax.broadcasted_iota(jnp.int32, (N_TOK, N_EXP), 1)
        onehot = (r == cols).astype(jnp.bfloat16)
        BLK = 128
        ri = lax.broadcasted_iota(jnp.int32, (BLK, BLK), 0)
        ci = lax.broadcasted_iota(jnp.int32, (BLK, BLK), 1)
        tril = (ci < ri).astype(jnp.bfloat16)
        offset = jnp.zeros((1, N_EXP), jnp.float32)
        pos_parts = []
        for b in range(N_TOK // BLK):
            oh_b = onehot[b * BLK:(b + 1) * BLK]
            pos_b = jnp.dot(tril, oh_b,
                            preferred_element_type=jnp.float32) + offset
            ohf = oh_b.astype(jnp.float32)
            pos_parts.append(jnp.sum(pos_b * ohf, axis=1, keepdims=True))
            offset = offset + jnp.sum(ohf, axis=0, keepdims=True)
        pos_tok = jnp.concatenate(pos_parts, axis=0)
        pos_ref[:, :] = pos_tok

        s_le = lax.broadcasted_iota(jnp.int32, (N_TOK, NC), 1) // CAP
        s_p = lax.broadcasted_iota(jnp.int32, (N_TOK, NC), 1) % CAP
        pt = jnp.logical_and(r == my * E_LOCAL + s_le,
                             pos_tok == s_p.astype(jnp.float32))
        ptb = pt.astype(jnp.bfloat16)
        xb = x_ref[:, :].astype(jnp.bfloat16)
        xg = lax.dot_general(
            ptb, xb, (((0,), (0,)), ((), ())),
            preferred_element_type=jnp.float32).astype(jnp.bfloat16)
        for le in range(E_LOCAL):
            c_blk = jnp.dot(
                xg[le * CAP:(le + 1) * CAP, :],
                w_ref[le].astype(jnp.bfloat16),
                preferred_element_type=jnp.float32)
            c_ref[pl.ds(le * CAP, CAP), :, :] = (
                c_blk.astype(jnp.bfloat16)[:, None, :])

        tok_iota = lax.broadcasted_iota(jnp.int32, (N_TOK, 1), 0)
        tok_cols = jnp.concatenate(
            [tok_iota // 32, tok_iota % 32,
             jnp.ones((N_TOK, 1), jnp.int32)], axis=1).astype(jnp.bfloat16)
        hlv = lax.dot_general(
            ptb, tok_cols, (((0,), (0,)), ((), ())),
            preferred_element_type=jnp.float32)
        t_vec = hlv[:, 0:1] * 32.0 + hlv[:, 1:2]
        meta_ref[:, :] = jnp.concatenate(
            [t_vec, hlv[:, 2:3]], axis=1).astype(jnp.int32)
        meta_copy = pltpu.make_async_copy(meta_ref, smem_ref, copy_sem)
        meta_copy.start()
        meta_copy.wait()

        for s in range(NC):
            t = smem_ref[s, 0]
            valid = smem_ref[s, 1]

            @pl.when(valid > 0)
            def _(s=s, t=t):
                dst_rank = lax.div(t, ROWS)
                dst_slot = lax.rem(t, ROWS)
                rdma = pltpu.make_async_remote_copy(
                    src_ref=c_ref.at[s],
                    dst_ref=stage_ref.at[dst_slot],
                    send_sem=send_sem,
                    recv_sem=recv_sem,
                    device_id=(dst_rank,),
                    device_id_type=pl.DeviceIdType.MESH,
                )
                rdma.start()

        pos_mine = pos_ref[pl.ds(my * ROWS, ROWS), :]
        keep_mine = (pos_mine < CAP).astype(jnp.float32)
        n_expect = jnp.sum(keep_mine).astype(jnp.int32)

        dummy = pltpu.make_async_remote_copy(
            src_ref=c_ref.at[0],
            dst_ref=stage_ref.at[0],
            send_sem=send_sem,
            recv_sem=recv_sem,
            device_id=(my,),
            device_id_type=pl.DeviceIdType.MESH,
        )

        def _wait_one(_, carry):
            dummy.wait_recv()
            return carry

        lax.fori_loop(0, n_expect, _wait_one, 0)

        out_ref[:, :] = jnp.where(
            keep_mine > 0, stage_ref[:, 0, :].astype(jnp.float32), 0.0)

        for s in range(NC):
            valid = smem_ref[s, 1]

            @pl.when(valid > 0)
            def _():
                dummy.wait_send()

    return pl.pallas_call(
        body,
        out_shape=jax.ShapeDtypeStruct((ROWS, D_OUT), jnp.float32),
        in_specs=[
            pl.BlockSpec(memory_space=pltpu.VMEM),
            pl.BlockSpec(memory_space=pltpu.VMEM),
            pl.BlockSpec(memory_space=pltpu.VMEM),
        ],
        out_specs=pl.BlockSpec(memory_space=pltpu.VMEM),
        scratch_shapes=[
            pltpu.VMEM((NC, 1, D_OUT), jnp.bfloat16),
            pltpu.VMEM((ROWS, 1, D_OUT), jnp.bfloat16),
            pltpu.VMEM((NC, 2), jnp.int32),
            pltpu.SMEM((NC, 2), jnp.int32),
            pltpu.VMEM((N_TOK, 1), jnp.float32),
            pltpu.SemaphoreType.DMA,
            pltpu.SemaphoreType.DMA,
            pltpu.SemaphoreType.DMA,
        ],
        compiler_params=pltpu.CompilerParams(collective_id=0),
    )(x, route_idx, expert_W)


# device time: 19244 ns/iter; 1.0989x vs baseline; 1.0523x over previous
import jax
import jax.numpy as jnp
from jax import lax
from jax.experimental import pallas as pl
from jax.experimental.pallas import tpu as pltpu

N_DEV = 16
N_TOK = 1024
D_IN = 512
D_OUT = 1024
N_EXP = 64
E_LOCAL = 4
CAP = 12
ROWS = N_TOK // N_DEV
NC = E_LOCAL * CAP


def kernel(x, router_W, route_idx, expert_W):
    del router_W

    def body(x_ref, idx_ref, w_ref, out_ref, c_ref, stage_ref, meta_ref,
             smem_ref, pos_ref, copy_sem, send_sem, recv_sem):
        my = lax.axis_index("i")

        barrier_sem = pltpu.get_barrier_semaphore()
        for d in range(N_DEV):
            pl.semaphore_signal(
                barrier_sem, inc=1,
                device_id=(d,), device_id_type=pl.DeviceIdType.MESH,
            )
        pl.semaphore_wait(barrier_sem, N_DEV)

        r = idx_ref[:, :]
        cols = lax.broadcasted_iota(jnp.int32, (N_TOK, N_EXP), 1)
        onehot = (r == cols).astype(jnp.bfloat16)
        BLK = 128
        ri = lax.broadcasted_iota(jnp.int32, (BLK, BLK), 0)
        ci = lax.broadcasted_iota(jnp.int32, (BLK, BLK), 1)
        tril = (ci < ri).astype(jnp.bfloat16)
        offset = jnp.zeros((1, N_EXP), jnp.float32)
        pos_parts = []
        for b in range(N_TOK // BLK):
            oh_b = onehot[b * BLK:(b + 1) * BLK]
            pos_b = jnp.dot(tril, oh_b,
                            preferred_element_type=jnp.float32) + offset
            ohf = oh_b.astype(jnp.float32)
            pos_parts.append(jnp.sum(pos_b * ohf, axis=1, keepdims=True))
            offset = offset + jnp.sum(ohf, axis=0, keepdims=True)
        pos_tok = jnp.concatenate(pos_parts, axis=0)
        pos_ref[:, :] = pos_tok

        s_le = lax.broadcasted_iota(jnp.int32, (N_TOK, NC), 1) // CAP
        s_p = lax.broadcasted_iota(jnp.int32, (N_TOK, NC), 1) % CAP
        pt = jnp.logical_and(r == my * E_LOCAL + s_le,
                             pos_tok == s_p.astype(jnp.float32))
        ptb = pt.astype(jnp.bfloat16)

        tok_iota = lax.broadcasted_iota(jnp.int32, (N_TOK, 1), 0)
        tok_cols = jnp.concatenate(
            [tok_iota // 32, tok_iota % 32,
             jnp.ones((N_TOK, 1), jnp.int32)], axis=1).astype(jnp.bfloat16)
        hlv = lax.dot_general(
            ptb, tok_cols, (((0,), (0,)), ((), ())),
            preferred_element_type=jnp.float32)
        t_vec = hlv[:, 0:1] * 32.0 + hlv[:, 1:2]
        meta_ref[:, :] = jnp.concatenate(
            [t_vec, hlv[:, 2:3]], axis=1).astype(jnp.int32)
        meta_copy = pltpu.make_async_copy(meta_ref, smem_ref, copy_sem)
        meta_copy.start()

        xb = x_ref[:, :].astype(jnp.bfloat16)
        xg = lax.dot_general(
            ptb, xb, (((0,), (0,)), ((), ())),
            preferred_element_type=jnp.float32).astype(jnp.bfloat16)

        meta_copy.wait()

        for le in range(E_LOCAL):
            c_blk = jnp.dot(
                xg[le * CAP:(le + 1) * CAP, :],
                w_ref[le].astype(jnp.bfloat16),
                preferred_element_type=jnp.float32)
            c_ref[pl.ds(le * CAP, CAP), :, :] = (
                c_blk.astype(jnp.bfloat16)[:, None, :])
            for s in range(le * CAP, (le + 1) * CAP):
                t = smem_ref[s, 0]
                valid = smem_ref[s, 1]

                @pl.when(valid > 0)
                def _(s=s, t=t):
                    dst_rank = lax.div(t, ROWS)
                    dst_slot = lax.rem(t, ROWS)
                    rdma = pltpu.make_async_remote_copy(
                        src_ref=c_ref.at[s],
                        dst_ref=stage_ref.at[dst_slot],
                        send_sem=send_sem,
                        recv_sem=recv_sem,
                        device_id=(dst_rank,),
                        device_id_type=pl.DeviceIdType.MESH,
                    )
                    rdma.start()

        pos_mine = pos_ref[pl.ds(my * ROWS, ROWS), :]
        keep_mine = (pos_mine < CAP).astype(jnp.float32)
        n_expect = jnp.sum(keep_mine).astype(jnp.int32)

        dummy = pltpu.make_async_remote_copy(
            src_ref=c_ref.at[0],
            dst_ref=stage_ref.at[0],
            send_sem=send_sem,
            recv_sem=recv_sem,
            device_id=(my,),
            device_id_type=pl.DeviceIdType.MESH,
        )

        def _wait_one(_, carry):
            dummy.wait_recv()
            return carry

        lax.fori_loop(0, n_expect, _wait_one, 0)

        out_ref[:, :] = jnp.where(
            keep_mine > 0, stage_ref[:, 0, :].astype(jnp.float32), 0.0)

        for s in range(NC):
            valid = smem_ref[s, 1]

            @pl.when(valid > 0)
            def _():
                dummy.wait_send()

    return pl.pallas_call(
        body,
        out_shape=jax.ShapeDtypeStruct((ROWS, D_OUT), jnp.float32),
        in_specs=[
            pl.BlockSpec(memory_space=pltpu.VMEM),
            pl.BlockSpec(memory_space=pltpu.VMEM),
            pl.BlockSpec(memory_space=pltpu.VMEM),
        ],
        out_specs=pl.BlockSpec(memory_space=pltpu.VMEM),
        scratch_shapes=[
            pltpu.VMEM((NC, 1, D_OUT), jnp.bfloat16),
            pltpu.VMEM((ROWS, 1, D_OUT), jnp.bfloat16),
            pltpu.VMEM((NC, 2), jnp.int32),
            pltpu.SMEM((NC, 2), jnp.int32),
            pltpu.VMEM((N_TOK, 1), jnp.float32),
            pltpu.SemaphoreType.DMA,
            pltpu.SemaphoreType.DMA,
            pltpu.SemaphoreType.DMA,
        ],
        compiler_params=pltpu.CompilerParams(collective_id=0),
    )(x, route_idx, expert_W)
